# baseline (speedup 1.0000x reference)
import functools
import math

import jax
import jax.numpy as jnp
from jax import lax
from jax.experimental import pallas as pl
from jax.experimental.pallas import tpu as pltpu
from jax.experimental.pallas import tpu_sc as plsc

N_ATOMS = 50000
N_PAD = 51200
N_SPECIES = 4
NCORE = 2
NSUB = 16
NWORK = NCORE * NSUB
EDGES_PER_WORKER = 25600
E_PAD = NWORK * EDGES_PER_WORKER
CHUNK = 2560
NCHUNKS = EDGES_PER_WORKER // CHUNK
NCH0 = 7
NCH1 = 2 * NCHUNKS - NCH0
BLK = 512
NBLK = CHUNK // BLK
ROWS_PER_TILE = N_PAD // NSUB

COEFF = math.sqrt(3.0 / (4.0 * math.pi))
PI = math.pi
HALF_PI = 0.5 * math.pi
CUT = 5.0
INNER = 4.5


def _sinp(u):
    u2 = u * u
    return u * (0.9999999660 + u2 * (-0.1666665066 + u2 * (
        0.0083330253 + u2 * (-0.0001980741 + u2 * 2.6019031e-06))))


def _cosp(u):
    u2 = u * u
    return 1.0 + u2 * (-0.4999999963 + u2 * (
        0.0416666418 + u2 * (-0.0013888397 + u2 * 2.4760495e-05)))


def _vgather(vec, idx):
    dnums = lax.GatherDimensionNumbers(
        offset_dims=(), collapsed_slice_dims=(0,), start_index_map=(0,))
    return lax.gather(vec, idx[:, None], dnums, (1,),
                      mode=lax.GatherScatterMode.PROMISE_IN_BOUNDS)


_mesh = plsc.VectorSubcoreMesh(core_axis_name="c", subcore_axis_name="s",
                               num_cores=NCORE, num_subcores=NSUB)

_ACC_SCRATCH = [pltpu.VMEM_SHARED((N_PAD,), jnp.float32) for _ in range(9)]


@functools.partial(
    pl.kernel,
    out_type=jax.ShapeDtypeStruct((NCORE * 9 * N_PAD,), jnp.float32),
    mesh=_mesh,
    scratch_types=[
        pltpu.VMEM((CHUNK,), jnp.float32),
        pltpu.VMEM((CHUNK,), jnp.float32),
        pltpu.VMEM((CHUNK,), jnp.float32),
        [pltpu.VMEM((BLK,), jnp.int32) for _ in range(4)],
        [pltpu.VMEM((BLK,), jnp.int32) for _ in range(4)],
        [pltpu.VMEM((BLK,), jnp.int32) for _ in range(4)],
        [pltpu.VMEM((BLK,), jnp.int32) for _ in range(4)],
        pltpu.VMEM((384,), jnp.float32),
        [[pltpu.VMEM((BLK,), jnp.float32) for _ in range(9)]
         for _ in range(4)],
        pltpu.VMEM_SHARED((N_ATOMS,), jnp.int32),
        _ACC_SCRATCH,
        pltpu.SemaphoreType.DMA,
        [pltpu.SemaphoreType.DMA for _ in range(4)],
        [pltpu.SemaphoreType.DMA for _ in range(4)],
        [pltpu.SemaphoreType.DMA for _ in range(4)],
    ],
)
def _sc_spex(vx_h, vy_h, vz_h, cen2_h, nbr2_h, spe_h, mt_h, zacc_h,
             out_h, vxb, vyb, vzb, cbufs, nbufs, spcbs, spnbs, mt, cqs,
             spes, accs, sem, semc, semg, sems):
    sid = lax.axis_index("s")
    cid = lax.axis_index("c")
    wid = cid * NSUB + sid

    pltpu.sync_copy(mt_h, mt)

    @pl.when(sid == 0)
    def _():
        pltpu.sync_copy(spe_h, spes)
    rbase = pl.multiple_of(sid * ROWS_PER_TILE, 128)
    for q in range(9):
        pltpu.sync_copy(zacc_h, accs[q].at[pl.ds(rbase, ROWS_PER_TILE)])
    plsc.subcore_barrier()

    mv = [mt[pl.ds(t * 16, 16)] for t in range(24)]

    tile_base = jnp.where(cid == 0, sid * NCH0,
                          NSUB * NCH0 + sid * NCH1) * CHUNK
    nch = jnp.where(cid == 0, NCH0, NCH1)

    def chunk_body(k, carry):
        base = pl.multiple_of(tile_base + k * CHUNK, CHUNK)
        pend_cn = [None] * 4
        pend_g = [None] * 4
        pend_sc = [None] * 4

        def fire_cn(bi):
            b = bi % 4
            bb = pl.multiple_of(base + bi * BLK, BLK)
            pend_cn[b] = (
                pltpu.async_copy(cen2_h.at[pl.ds(bb, BLK)], cbufs[b], semc[b]),
                pltpu.async_copy(nbr2_h.at[pl.ds(bb, BLK)], nbufs[b], semc[b]))

        def fire_g(bi):
            b = bi % 4
            for d in pend_cn[b]:
                d.wait()
            pend_g[b] = (
                pltpu.async_copy(spes.at[cbufs[b]], spcbs[b], semg[b]),
                pltpu.async_copy(spes.at[nbufs[b]], spnbs[b], semg[b]))

        dx = pltpu.async_copy(vx_h.at[pl.ds(base, CHUNK)], vxb, sem)
        dy = pltpu.async_copy(vy_h.at[pl.ds(base, CHUNK)], vyb, sem)
        dz = pltpu.async_copy(vz_h.at[pl.ds(base, CHUNK)], vzb, sem)
        fire_cn(0)
        fire_cn(1)
        fire_g(0)
        dx.wait()
        dy.wait()
        dz.wait()

        for bi in range(NBLK):
            b = bi % 4
            for d in pend_g[b]:
                d.wait()
            if bi + 2 < NBLK:
                b2 = (bi + 2) % 4
                if pend_sc[b2] is not None:
                    for d in pend_sc[b2]:
                        d.wait()
                    pend_sc[b2] = None
                fire_cn(bi + 2)
            if bi + 1 < NBLK:
                fire_g(bi + 1)

            def vec_body(l, c_, bi=bi, b=b):
                off = bi * BLK + l * 16
                x = vxb[pl.ds(off, 16)]
                y = vyb[pl.ds(off, 16)]
                z = vzb[pl.ds(off, 16)]
                r2 = x * x + y * y + z * z + 1e-12
                yi = jnp.int32(0x5F3759DF) - (
                    lax.bitcast_convert_type(r2, jnp.int32) >> 1)
                ys = lax.bitcast_convert_type(yi, jnp.float32)
                ys = ys * (1.5 - 0.5 * r2 * ys * ys)
                ys = ys * (1.5 - 0.5 * r2 * ys * ys)
                ys = ys * (1.5 - 0.5 * r2 * ys * ys)
                r = r2 * ys
                u = (PI / CUT) * jnp.minimum(r, CUT) - HALF_PI
                s1 = _cosp(u)
                c2 = -2.0 * _sinp(u)
                phi = jnp.clip(2.0 * PI * (r - INNER), 0.0, PI) - HALF_PI
                fcm = 0.5 * (1.0 - _sinp(phi))
                fc = jnp.where(r < INNER, 1.0, jnp.where(r < CUT, fcm, 0.0))
                gr = fc * ys
                t1 = c2 * s1
                ts = [s1 * gr, t1 * gr]
                sm2, sm1 = s1, t1
                for _ in range(6):
                    sn = c2 * sm1 - sm2
                    ts.append(sn * gr)
                    sm2, sm1 = sm1, sn
                si = (spcbs[b][pl.ds(l * 16, 16)] * N_SPECIES
                      + spnbs[b][pl.ds(l * 16, 16)])
                bs = []
                for j in range(3):
                    bj = ts[0] * _vgather(mv[j * 8], si)
                    for n in range(1, 8):
                        bj = bj + ts[n] * _vgather(mv[j * 8 + n], si)
                    bs.append(bj)
                ys_c = COEFF * ys
                yv = [ys_c * y, ys_c * z, ys_c * x]
                for m in range(3):
                    for j in range(3):
                        cqs[b][m * 3 + j][pl.ds(l * 16, 16)] = yv[m] * bs[j]
                return c_

            lax.fori_loop(0, BLK // 16, vec_body, 0)
            pend_sc[b] = [pltpu.async_copy(cqs[b][q], accs[q].at[cbufs[b]],
                                           sems[b], add=True)
                          for q in range(9)]
        for p in pend_sc:
            if p is not None:
                for d in p:
                    d.wait()
        return carry

    lax.fori_loop(0, nch, chunk_body, 0)

    plsc.subcore_barrier()
    for q in range(9):
        obase = pl.multiple_of((cid * 9 + q) * N_PAD + rbase, 128)
        pltpu.sync_copy(accs[q].at[pl.ds(rbase, ROWS_PER_TILE)],
                        out_h.at[pl.ds(obase, ROWS_PER_TILE)])


def _combine_body(a_ref, o_ref):
    o_ref[...] = a_ref[0] + a_ref[1]


def _combine(partials):
    a = partials.reshape(NCORE, 9 * N_PAD // 128, 128)
    return pl.pallas_call(
        _combine_body,
        out_shape=jax.ShapeDtypeStruct((9 * N_PAD // 128, 128), jnp.float32),
    )(a)


def kernel(interatomic_vectors, centers, neighbors, species, structures,
           atom_index_in_structure, W_alch, center_embedding, W_contract):
    e = interatomic_vectors.shape[0]
    pad = E_PAD - e
    vx = jnp.pad(interatomic_vectors[:, 0], (0, pad))
    vy = jnp.pad(interatomic_vectors[:, 1], (0, pad))
    vz = jnp.pad(interatomic_vectors[:, 2], (0, pad))
    cen2 = jnp.pad(centers, (0, pad))
    nbr2 = jnp.pad(neighbors, (0, pad))
    t = (center_embedding[:, None, :] * W_contract[None, :, :]).reshape(
        N_SPECIES, 3, N_SPECIES, 8)
    m = jnp.einsum('sp,cjpn->jncs', W_alch, t).reshape(24, 16).reshape(-1)
    zacc = jnp.zeros((ROWS_PER_TILE,), jnp.float32)
    partials = _sc_spex(vx, vy, vz, cen2, nbr2, species, m, zacc)
    comb = _combine(partials)
    comb = comb.reshape(9, N_PAD)[:, :N_ATOMS]
    return jnp.transpose(comb).reshape(N_ATOMS, 3, 3)


# speedup vs baseline: 1.2372x; 1.2372x over previous
import functools
import math

import jax
import jax.numpy as jnp
from jax import lax
from jax.experimental import pallas as pl
from jax.experimental.pallas import tpu as pltpu
from jax.experimental.pallas import tpu_sc as plsc

N_ATOMS = 50000
N_PAD = 51200
N_SPECIES = 4
NCORE = 2
NSUB = 16
NWORK = NCORE * NSUB
EDGES_PER_WORKER = 25600
E_PAD = NWORK * EDGES_PER_WORKER
CHUNK = 2560
NCHUNKS = EDGES_PER_WORKER // CHUNK
NCH0 = 13
NCH1 = 2 * NCHUNKS - NCH0
BLK = 512
NBLK = CHUNK // BLK
ROWS_PER_TILE = N_PAD // NSUB

COEFF = math.sqrt(3.0 / (4.0 * math.pi))
PI = math.pi
HALF_PI = 0.5 * math.pi
CUT = 5.0
INNER = 4.5


def _sinp(u):
    u2 = u * u
    return u * (0.9999999660 + u2 * (-0.1666665066 + u2 * (
        0.0083330253 + u2 * (-0.0001980741 + u2 * 2.6019031e-06))))


def _cosp(u):
    u2 = u * u
    return 1.0 + u2 * (-0.4999999963 + u2 * (
        0.0416666418 + u2 * (-0.0013888397 + u2 * 2.4760495e-05)))


def _vgather(vec, idx):
    dnums = lax.GatherDimensionNumbers(
        offset_dims=(), collapsed_slice_dims=(0,), start_index_map=(0,))
    return lax.gather(vec, idx[:, None], dnums, (1,),
                      mode=lax.GatherScatterMode.PROMISE_IN_BOUNDS)


_mesh = plsc.VectorSubcoreMesh(core_axis_name="c", subcore_axis_name="s",
                               num_cores=NCORE, num_subcores=NSUB)

_ACC_SCRATCH = [pltpu.VMEM_SHARED((N_PAD,), jnp.float32) for _ in range(9)]


@functools.partial(
    pl.kernel,
    out_type=jax.ShapeDtypeStruct((NCORE * 9 * N_PAD,), jnp.float32),
    mesh=_mesh,
    scratch_types=[
        pltpu.VMEM((CHUNK,), jnp.float32),
        pltpu.VMEM((CHUNK,), jnp.float32),
        pltpu.VMEM((CHUNK,), jnp.float32),
        [pltpu.VMEM((BLK,), jnp.int32) for _ in range(4)],
        [pltpu.VMEM((BLK,), jnp.int32) for _ in range(4)],
        [pltpu.VMEM((BLK,), jnp.int32) for _ in range(4)],
        [pltpu.VMEM((BLK,), jnp.int32) for _ in range(4)],
        pltpu.VMEM((384,), jnp.float32),
        [[pltpu.VMEM((BLK,), jnp.float32) for _ in range(9)]
         for _ in range(4)],
        pltpu.VMEM_SHARED((N_ATOMS,), jnp.int32),
        _ACC_SCRATCH,
        pltpu.SemaphoreType.DMA,
        [pltpu.SemaphoreType.DMA for _ in range(4)],
        [pltpu.SemaphoreType.DMA for _ in range(4)],
        [pltpu.SemaphoreType.DMA for _ in range(4)],
    ],
)
def _sc_spex(vx_h, vy_h, vz_h, cen2_h, nbr2_h, spe_h, mt_h, zacc_h,
             out_h, vxb, vyb, vzb, cbufs, nbufs, spcbs, spnbs, mt, cqs,
             spes, accs, sem, semc, semg, sems):
    sid = lax.axis_index("s")
    cid = lax.axis_index("c")
    wid = cid * NSUB + sid

    pltpu.sync_copy(mt_h, mt)

    @pl.when(sid == 0)
    def _():
        pltpu.sync_copy(spe_h, spes)
    rbase = pl.multiple_of(sid * ROWS_PER_TILE, 128)
    for q in range(9):
        pltpu.sync_copy(zacc_h, accs[q].at[pl.ds(rbase, ROWS_PER_TILE)])
    plsc.subcore_barrier()

    mv = [mt[pl.ds(t * 16, 16)] for t in range(24)]

    tile_base = jnp.where(cid == 0, sid * NCH0,
                          NSUB * NCH0 + sid * NCH1) * CHUNK
    nch = jnp.where(cid == 0, NCH0, NCH1)

    def chunk_body(k, carry):
        base = pl.multiple_of(tile_base + k * CHUNK, CHUNK)
        pend_cn = [None] * 4
        pend_g = [None] * 4
        pend_sc = [None] * 4

        def fire_cn(bi):
            b = bi % 4
            bb = pl.multiple_of(base + bi * BLK, BLK)
            pend_cn[b] = (
                pltpu.async_copy(cen2_h.at[pl.ds(bb, BLK)], cbufs[b], semc[b]),
                pltpu.async_copy(nbr2_h.at[pl.ds(bb, BLK)], nbufs[b], semc[b]))

        def fire_g(bi):
            b = bi % 4
            for d in pend_cn[b]:
                d.wait()
            pend_g[b] = (
                pltpu.async_copy(spes.at[cbufs[b]], spcbs[b], semg[b]),
                pltpu.async_copy(spes.at[nbufs[b]], spnbs[b], semg[b]))

        dx = pltpu.async_copy(vx_h.at[pl.ds(base, CHUNK)], vxb, sem)
        dy = pltpu.async_copy(vy_h.at[pl.ds(base, CHUNK)], vyb, sem)
        dz = pltpu.async_copy(vz_h.at[pl.ds(base, CHUNK)], vzb, sem)
        fire_cn(0)
        fire_cn(1)
        fire_g(0)
        dx.wait()
        dy.wait()
        dz.wait()

        for bi in range(NBLK):
            b = bi % 4
            for d in pend_g[b]:
                d.wait()
            if bi + 2 < NBLK:
                b2 = (bi + 2) % 4
                if pend_sc[b2] is not None:
                    for d in pend_sc[b2]:
                        d.wait()
                    pend_sc[b2] = None
                fire_cn(bi + 2)
            if bi + 1 < NBLK:
                fire_g(bi + 1)

            def vec_body(l, c_, bi=bi, b=b):
                off = bi * BLK + l * 16
                x = vxb[pl.ds(off, 16)]
                y = vyb[pl.ds(off, 16)]
                z = vzb[pl.ds(off, 16)]
                r2 = x * x + y * y + z * z + 1e-12
                yi = jnp.int32(0x5F3759DF) - (
                    lax.bitcast_convert_type(r2, jnp.int32) >> 1)
                ys = lax.bitcast_convert_type(yi, jnp.float32)
                ys = ys * (1.5 - 0.5 * r2 * ys * ys)
                ys = ys * (1.5 - 0.5 * r2 * ys * ys)
                ys = ys * (1.5 - 0.5 * r2 * ys * ys)
                r = r2 * ys
                u = (PI / CUT) * jnp.minimum(r, CUT) - HALF_PI
                s1 = _cosp(u)
                c2 = -2.0 * _sinp(u)
                phi = jnp.clip(2.0 * PI * (r - INNER), 0.0, PI) - HALF_PI
                fcm = 0.5 * (1.0 - _sinp(phi))
                fc = jnp.where(r < INNER, 1.0, jnp.where(r < CUT, fcm, 0.0))
                gr = fc * ys
                t1 = c2 * s1
                ts = [s1 * gr, t1 * gr]
                sm2, sm1 = s1, t1
                for _ in range(6):
                    sn = c2 * sm1 - sm2
                    ts.append(sn * gr)
                    sm2, sm1 = sm1, sn
                si = (spcbs[b][pl.ds(l * 16, 16)] * N_SPECIES
                      + spnbs[b][pl.ds(l * 16, 16)])
                bs = []
                for j in range(3):
                    bj = ts[0] * _vgather(mv[j * 8], si)
                    for n in range(1, 8):
                        bj = bj + ts[n] * _vgather(mv[j * 8 + n], si)
                    bs.append(bj)
                ys_c = COEFF * ys
                yv = [ys_c * y, ys_c * z, ys_c * x]
                for m in range(3):
                    for j in range(3):
                        cqs[b][m * 3 + j][pl.ds(l * 16, 16)] = yv[m] * bs[j]
                return c_

            lax.fori_loop(0, BLK // 16, vec_body, 0)
            pend_sc[b] = [pltpu.async_copy(cqs[b][q], accs[q].at[cbufs[b]],
                                           sems[b], add=True)
                          for q in range(9)]
        for p in pend_sc:
            if p is not None:
                for d in p:
                    d.wait()
        return carry

    lax.fori_loop(0, nch, chunk_body, 0)

    plsc.subcore_barrier()
    for q in range(9):
        obase = pl.multiple_of((cid * 9 + q) * N_PAD + rbase, 128)
        pltpu.sync_copy(accs[q].at[pl.ds(rbase, ROWS_PER_TILE)],
                        out_h.at[pl.ds(obase, ROWS_PER_TILE)])


def _combine_body(a_ref, o_ref):
    o_ref[...] = a_ref[0] + a_ref[1]


def _combine(partials):
    a = partials.reshape(NCORE, 9 * N_PAD // 128, 128)
    return pl.pallas_call(
        _combine_body,
        out_shape=jax.ShapeDtypeStruct((9 * N_PAD // 128, 128), jnp.float32),
    )(a)


def kernel(interatomic_vectors, centers, neighbors, species, structures,
           atom_index_in_structure, W_alch, center_embedding, W_contract):
    e = interatomic_vectors.shape[0]
    pad = E_PAD - e
    vx = jnp.pad(interatomic_vectors[:, 0], (0, pad))
    vy = jnp.pad(interatomic_vectors[:, 1], (0, pad))
    vz = jnp.pad(interatomic_vectors[:, 2], (0, pad))
    cen2 = jnp.pad(centers, (0, pad))
    nbr2 = jnp.pad(neighbors, (0, pad))
    t = (center_embedding[:, None, :] * W_contract[None, :, :]).reshape(
        N_SPECIES, 3, N_SPECIES, 8)
    m = jnp.einsum('sp,cjpn->jncs', W_alch, t).reshape(24, 16).reshape(-1)
    zacc = jnp.zeros((ROWS_PER_TILE,), jnp.float32)
    partials = _sc_spex(vx, vy, vz, cen2, nbr2, species, m, zacc)
    comb = _combine(partials)
    comb = comb.reshape(9, N_PAD)[:, :N_ATOMS]
    return jnp.transpose(comb).reshape(N_ATOMS, 3, 3)


# speedup vs baseline: 1.2863x; 1.0397x over previous
import functools
import math

import jax
import jax.numpy as jnp
from jax import lax
from jax.experimental import pallas as pl
from jax.experimental.pallas import tpu as pltpu
from jax.experimental.pallas import tpu_sc as plsc

N_ATOMS = 50000
N_PAD = 51200
N_SPECIES = 4
NCORE = 2
NSUB = 16
NWORK = NCORE * NSUB
EDGES_PER_WORKER = 25600
E_PAD = NWORK * EDGES_PER_WORKER
CHUNK = 2560
NCHUNKS = EDGES_PER_WORKER // CHUNK
NCH0 = 13
NCH1 = 2 * NCHUNKS - NCH0
BLK = 512
NBLK = CHUNK // BLK
ROWS_PER_TILE = N_PAD // NSUB

COEFF = math.sqrt(3.0 / (4.0 * math.pi))
PI = math.pi
HALF_PI = 0.5 * math.pi
CUT = 5.0
INNER = 4.5


def _sinp(u):
    u2 = u * u
    return u * (0.9999999660 + u2 * (-0.1666665066 + u2 * (
        0.0083330253 + u2 * (-0.0001980741 + u2 * 2.6019031e-06))))


def _cosp(u):
    u2 = u * u
    return 1.0 + u2 * (-0.4999999963 + u2 * (
        0.0416666418 + u2 * (-0.0013888397 + u2 * 2.4760495e-05)))


def _vgather(vec, idx):
    dnums = lax.GatherDimensionNumbers(
        offset_dims=(), collapsed_slice_dims=(0,), start_index_map=(0,))
    return lax.gather(vec, idx[:, None], dnums, (1,),
                      mode=lax.GatherScatterMode.PROMISE_IN_BOUNDS)


_mesh = plsc.VectorSubcoreMesh(core_axis_name="c", subcore_axis_name="s",
                               num_cores=NCORE, num_subcores=NSUB)

_ACC_SCRATCH = [pltpu.VMEM_SHARED((N_PAD,), jnp.float32) for _ in range(9)]


@functools.partial(
    pl.kernel,
    out_type=jax.ShapeDtypeStruct((NCORE * 9 * N_PAD,), jnp.float32),
    mesh=_mesh,
    scratch_types=[
        pltpu.VMEM((CHUNK,), jnp.float32),
        pltpu.VMEM((CHUNK,), jnp.float32),
        pltpu.VMEM((CHUNK,), jnp.float32),
        [pltpu.VMEM((BLK,), jnp.int32) for _ in range(4)],
        [pltpu.VMEM((BLK,), jnp.int32) for _ in range(4)],
        [pltpu.VMEM((BLK,), jnp.int32) for _ in range(4)],
        [pltpu.VMEM((BLK,), jnp.int32) for _ in range(4)],
        pltpu.VMEM((384,), jnp.float32),
        pltpu.VMEM((ROWS_PER_TILE,), jnp.float32),
        [[pltpu.VMEM((BLK,), jnp.float32) for _ in range(9)]
         for _ in range(4)],
        pltpu.VMEM_SHARED((N_ATOMS,), jnp.int32),
        _ACC_SCRATCH,
        pltpu.SemaphoreType.DMA,
        [pltpu.SemaphoreType.DMA for _ in range(4)],
        [pltpu.SemaphoreType.DMA for _ in range(4)],
        [pltpu.SemaphoreType.DMA for _ in range(4)],
    ],
)
def _sc_spex(vx_h, vy_h, vz_h, cen2_h, nbr2_h, spe_h, mt_h,
             out_h, vxb, vyb, vzb, cbufs, nbufs, spcbs, spnbs, mt, zbuf,
             cqs, spes, accs, sem, semc, semg, sems):
    sid = lax.axis_index("s")
    cid = lax.axis_index("c")
    wid = cid * NSUB + sid

    pltpu.sync_copy(mt_h, mt)

    @pl.when(sid == 0)
    def _():
        pltpu.sync_copy(spe_h, spes)
    zv16 = jnp.zeros((16,), jnp.float32)

    def zbody(i, c_):
        zbuf[pl.ds(i * 16, 16)] = zv16
        return c_
    lax.fori_loop(0, ROWS_PER_TILE // 16, zbody, 0)
    rbase = pl.multiple_of(sid * ROWS_PER_TILE, 128)
    zw = [pltpu.async_copy(zbuf, accs[q].at[pl.ds(rbase, ROWS_PER_TILE)],
                           sem) for q in range(9)]
    for d in zw:
        d.wait()
    plsc.subcore_barrier()

    mv = [mt[pl.ds(t * 16, 16)] for t in range(24)]

    tile_base = jnp.where(cid == 0, sid * NCH0,
                          NSUB * NCH0 + sid * NCH1) * CHUNK
    nch = jnp.where(cid == 0, NCH0, NCH1)

    def chunk_body(k, carry):
        base = pl.multiple_of(tile_base + k * CHUNK, CHUNK)
        pend_cn = [None] * 4
        pend_g = [None] * 4
        pend_sc = [None] * 4

        def fire_cn(bi):
            b = bi % 4
            bb = pl.multiple_of(base + bi * BLK, BLK)
            pend_cn[b] = (
                pltpu.async_copy(cen2_h.at[pl.ds(bb, BLK)], cbufs[b], semc[b]),
                pltpu.async_copy(nbr2_h.at[pl.ds(bb, BLK)], nbufs[b], semc[b]))

        def fire_g(bi):
            b = bi % 4
            for d in pend_cn[b]:
                d.wait()
            pend_g[b] = (
                pltpu.async_copy(spes.at[cbufs[b]], spcbs[b], semg[b]),
                pltpu.async_copy(spes.at[nbufs[b]], spnbs[b], semg[b]))

        dx = pltpu.async_copy(vx_h.at[pl.ds(base, CHUNK)], vxb, sem)
        dy = pltpu.async_copy(vy_h.at[pl.ds(base, CHUNK)], vyb, sem)
        dz = pltpu.async_copy(vz_h.at[pl.ds(base, CHUNK)], vzb, sem)
        fire_cn(0)
        fire_cn(1)
        fire_g(0)
        dx.wait()
        dy.wait()
        dz.wait()

        for bi in range(NBLK):
            b = bi % 4
            for d in pend_g[b]:
                d.wait()
            if bi + 2 < NBLK:
                b2 = (bi + 2) % 4
                if pend_sc[b2] is not None:
                    for d in pend_sc[b2]:
                        d.wait()
                    pend_sc[b2] = None
                fire_cn(bi + 2)
            if bi + 1 < NBLK:
                fire_g(bi + 1)

            def vec_body(l, c_, bi=bi, b=b):
                off = bi * BLK + l * 16
                x = vxb[pl.ds(off, 16)]
                y = vyb[pl.ds(off, 16)]
                z = vzb[pl.ds(off, 16)]
                r2 = x * x + y * y + z * z + 1e-12
                yi = jnp.int32(0x5F3759DF) - (
                    lax.bitcast_convert_type(r2, jnp.int32) >> 1)
                ys = lax.bitcast_convert_type(yi, jnp.float32)
                ys = ys * (1.5 - 0.5 * r2 * ys * ys)
                ys = ys * (1.5 - 0.5 * r2 * ys * ys)
                ys = ys * (1.5 - 0.5 * r2 * ys * ys)
                r = r2 * ys
                u = (PI / CUT) * jnp.minimum(r, CUT) - HALF_PI
                s1 = _cosp(u)
                c2 = -2.0 * _sinp(u)
                phi = jnp.clip(2.0 * PI * (r - INNER), 0.0, PI) - HALF_PI
                fcm = 0.5 * (1.0 - _sinp(phi))
                fc = jnp.where(r < INNER, 1.0, jnp.where(r < CUT, fcm, 0.0))
                gr = fc * ys
                ts = [s1, c2 * s1]
                sm2, sm1 = ts[0], ts[1]
                for _ in range(6):
                    sn = c2 * sm1 - sm2
                    ts.append(sn)
                    sm2, sm1 = sm1, sn
                si = (spcbs[b][pl.ds(l * 16, 16)] * N_SPECIES
                      + spnbs[b][pl.ds(l * 16, 16)])
                bs = []
                for j in range(3):
                    bj = ts[0] * _vgather(mv[j * 8], si)
                    for n in range(1, 8):
                        bj = bj + ts[n] * _vgather(mv[j * 8 + n], si)
                    bs.append(bj * gr)
                ys_c = COEFF * ys
                yv = [ys_c * y, ys_c * z, ys_c * x]
                for m in range(3):
                    for j in range(3):
                        cqs[b][m * 3 + j][pl.ds(l * 16, 16)] = yv[m] * bs[j]
                return c_

            lax.fori_loop(0, BLK // 16, vec_body, 0)
            pend_sc[b] = [pltpu.async_copy(cqs[b][q], accs[q].at[cbufs[b]],
                                           sems[b], add=True)
                          for q in range(9)]
        for p in pend_sc:
            if p is not None:
                for d in p:
                    d.wait()
        return carry

    lax.fori_loop(0, nch, chunk_body, 0)

    plsc.subcore_barrier()
    for q in range(9):
        obase = pl.multiple_of((cid * 9 + q) * N_PAD + rbase, 128)
        pltpu.sync_copy(accs[q].at[pl.ds(rbase, ROWS_PER_TILE)],
                        out_h.at[pl.ds(obase, ROWS_PER_TILE)])


def _combine_body(a_ref, o_ref):
    o_ref[...] = a_ref[0] + a_ref[1]


def _combine(partials):
    a = partials.reshape(NCORE, 9 * N_PAD // 128, 128)
    return pl.pallas_call(
        _combine_body,
        out_shape=jax.ShapeDtypeStruct((9 * N_PAD // 128, 128), jnp.float32),
    )(a)


def kernel(interatomic_vectors, centers, neighbors, species, structures,
           atom_index_in_structure, W_alch, center_embedding, W_contract):
    e = interatomic_vectors.shape[0]
    pad = E_PAD - e
    vx = jnp.pad(interatomic_vectors[:, 0], (0, pad))
    vy = jnp.pad(interatomic_vectors[:, 1], (0, pad))
    vz = jnp.pad(interatomic_vectors[:, 2], (0, pad))
    cen2 = jnp.pad(centers, (0, pad))
    nbr2 = jnp.pad(neighbors, (0, pad))
    t = (center_embedding[:, None, :] * W_contract[None, :, :]).reshape(
        N_SPECIES, 3, N_SPECIES, 8)
    m = jnp.einsum('sp,cjpn->jncs', W_alch, t).reshape(24, 16).reshape(-1)
    partials = _sc_spex(vx, vy, vz, cen2, nbr2, species, m)
    comb = _combine(partials)
    comb = comb.reshape(9, N_PAD)[:, :N_ATOMS]
    return jnp.transpose(comb).reshape(N_ATOMS, 3, 3)
